# Initial kernel scaffold; baseline (speedup 1.0000x reference)
import jax
import jax.numpy as jnp
from jax.experimental import pallas as pl


def kernel(inputs, kernel, bias):
    raise NotImplementedError("write your pallas kernel here")


# speedup vs baseline: 1.0407x; 1.0407x over previous
import jax
import jax.numpy as jnp
from jax.experimental import pallas as pl
from jax.experimental.pallas import tpu as pltpu

_C = 2048
_T = 128
_NJ = _C // _T
_BM = 512


def _band_body(x0, x1, x2, w0, w1, w2, b, o):
    di = jax.lax.broadcasted_iota(jnp.int32, (_T, _T), 0)
    dj = jax.lax.broadcasted_iota(jnp.int32, (_T, _T), 1)
    wl = jnp.where(di >= dj, w0[...], 0.0)
    wu = jnp.where(di < dj, w2[...], 0.0)
    acc = jnp.dot(x0[...], wl, preferred_element_type=jnp.float32)
    acc += jnp.dot(x1[...], w1[...], preferred_element_type=jnp.float32)
    acc += jnp.dot(x2[...], wu, preferred_element_type=jnp.float32)
    o[...] = acc + b[...]


def kernel(inputs, kernel, bias):
    batch = inputs.shape[0]
    nb = batch // _BM
    bias2 = bias.reshape(1, _C)
    grid = (_NJ, nb)

    out = pl.pallas_call(
        _band_body,
        grid=grid,
        in_specs=[
            pl.BlockSpec((_BM, _T), lambda j, b: (b, j)),
            pl.BlockSpec((_BM, _T), lambda j, b: (b, (j + 1) % _NJ)),
            pl.BlockSpec((_BM, _T), lambda j, b: (b, (j + 2) % _NJ)),
            pl.BlockSpec((_T, _T), lambda j, b: (j, j)),
            pl.BlockSpec((_T, _T), lambda j, b: ((j + 1) % _NJ, j)),
            pl.BlockSpec((_T, _T), lambda j, b: ((j + 2) % _NJ, j)),
            pl.BlockSpec((1, _T), lambda j, b: (0, j)),
        ],
        out_specs=pl.BlockSpec((_BM, _T), lambda j, b: (b, j)),
        out_shape=jax.ShapeDtypeStruct((batch, _C), jnp.float32),
        compiler_params=pltpu.CompilerParams(
            dimension_semantics=("arbitrary", "arbitrary"),
        ),
    )(inputs, inputs, inputs, kernel, kernel, kernel, bias2)
    return out


# speedup vs baseline: 1.6622x; 1.5972x over previous
import jax
import jax.numpy as jnp
from jax.experimental import pallas as pl
from jax.experimental.pallas import tpu as pltpu

_C = 2048
_T = 256
_NJ = _C // _T
_BM = 512


def _band_body(x, w0, w1, b, o):
    j = pl.program_id(1)
    c0 = x[:, pl.ds(j * _T, _T)].astype(jnp.bfloat16)
    c1 = x[:, pl.ds(((j + 1) % _NJ) * _T, _T)].astype(jnp.bfloat16)
    di = jax.lax.broadcasted_iota(jnp.int32, (_T, _T), 0)
    dj = jax.lax.broadcasted_iota(jnp.int32, (_T, _T), 1)
    wl = jnp.where(di >= dj, w0[...], 0.0).astype(jnp.bfloat16)
    wu = jnp.where(di < dj, w1[...], 0.0).astype(jnp.bfloat16)
    acc = jnp.dot(c0, wl, preferred_element_type=jnp.float32)
    acc += jnp.dot(c1, wu, preferred_element_type=jnp.float32)
    o[...] = acc + b[...]


def kernel(inputs, kernel, bias):
    batch = inputs.shape[0]
    nb = batch // _BM
    bias2 = bias.reshape(1, _C)
    grid = (nb, _NJ)

    out = pl.pallas_call(
        _band_body,
        grid=grid,
        in_specs=[
            pl.BlockSpec((_BM, _C), lambda b, j: (b, 0)),
            pl.BlockSpec((_T, _T), lambda b, j: (j, j)),
            pl.BlockSpec((_T, _T), lambda b, j: ((j + 1) % _NJ, j)),
            pl.BlockSpec((1, _T), lambda b, j: (0, j)),
        ],
        out_specs=pl.BlockSpec((_BM, _T), lambda b, j: (b, j)),
        out_shape=jax.ShapeDtypeStruct((batch, _C), jnp.float32),
        compiler_params=pltpu.CompilerParams(
            dimension_semantics=("arbitrary", "arbitrary"),
        ),
    )(inputs, kernel, kernel, bias2)
    return out
